# Initial kernel scaffold; baseline (speedup 1.0000x reference)
import jax
import jax.numpy as jnp
from jax.experimental import pallas as pl


def kernel(x, edge_index, edge_label_index, W1l, W1r, b1, W2l, W2r, b2):
    raise NotImplementedError("write your pallas kernel here")


# speedup vs baseline: 2.8657x; 2.8657x over previous
import functools

import jax
import jax.numpy as jnp
from jax import lax
from jax.experimental import pallas as pl
from jax.experimental.pallas import tpu as pltpu
from jax.experimental.pallas import tpu_sc as plsc

_N = 10000
_D = 128
_E = 320000
_EL = 100000

_NT = 32
_CH = 128
_EC = 79
_EP = _NT * _EC * _CH
_ROWS_PER_TILE = 640
_NP = 16 * _ROWS_PER_TILE
_DC = 25
_ELP = _NT * _DC * _CH

@functools.lru_cache(maxsize=None)
def _mesh():
    return plsc.VectorSubcoreMesh(core_axis_name="c", subcore_axis_name="s")


def _agg_body(with_count, *refs):
    if with_count:
        (x_hbm, src_hbm, dst_hbm, zr_hbm, zc_hbm, agg_out, cnt_out,
         sidx, didx, rows, ones, agg_sp, cnt_sp, sem) = refs
    else:
        (x_hbm, src_hbm, dst_hbm, zr_hbm, agg_out,
         sidx, didx, rows, agg_sp, sem) = refs
    cid = lax.axis_index("c")
    sid = lax.axis_index("s")
    wid = sid * 2 + cid
    rbase = sid * _ROWS_PER_TILE

    pltpu.sync_copy(zr_hbm, agg_sp.at[pl.ds(rbase, _ROWS_PER_TILE)])
    if with_count:
        pltpu.sync_copy(zc_hbm, cnt_sp.at[pl.ds(rbase, _ROWS_PER_TILE)])
        for i in range(_CH // 16):
            ones[pl.ds(i * 16, 16)] = jnp.ones((16,), jnp.float32)
    plsc.subcore_barrier()

    ebase = wid * (_EC * _CH)

    def chunk(k, carry):
        b = ebase + k * _CH
        pltpu.sync_copy(src_hbm.at[pl.ds(b, _CH)], sidx)
        pltpu.sync_copy(dst_hbm.at[pl.ds(b, _CH)], didx)
        pltpu.async_copy(x_hbm.at[sidx], rows, sem).wait()
        pltpu.sync_copy(rows, agg_sp.at[didx], add=True)
        if with_count:
            pltpu.sync_copy(ones, cnt_sp.at[didx], add=True)
        return carry

    lax.fori_loop(0, _EC, chunk, 0)
    plsc.subcore_barrier()

    pltpu.sync_copy(agg_sp.at[pl.ds(rbase, _ROWS_PER_TILE)],
                    agg_out.at[cid, pl.ds(rbase, _ROWS_PER_TILE)])
    if with_count:
        pltpu.sync_copy(cnt_sp.at[pl.ds(rbase, _ROWS_PER_TILE)],
                        cnt_out.at[cid, pl.ds(rbase, _ROWS_PER_TILE)])


@functools.lru_cache(maxsize=None)
def _agg_count_call():
    return functools.partial(
        pl.kernel,
        mesh=_mesh(),
        out_type=[jax.ShapeDtypeStruct((2, _NP, _D), jnp.float32),
                  jax.ShapeDtypeStruct((2, _NP), jnp.float32)],
        scratch_types=[
            pltpu.VMEM((_CH,), jnp.int32),
            pltpu.VMEM((_CH,), jnp.int32),
            pltpu.VMEM((_CH, _D), jnp.float32),
            pltpu.VMEM((_CH,), jnp.float32),
            pltpu.VMEM_SHARED((_NP, _D), jnp.float32),
            pltpu.VMEM_SHARED((_NP,), jnp.float32),
            pltpu.SemaphoreType.DMA,
        ],
    )(functools.partial(_agg_body, True))


@functools.lru_cache(maxsize=None)
def _agg_call():
    return functools.partial(
        pl.kernel,
        mesh=_mesh(),
        out_type=[jax.ShapeDtypeStruct((2, _NP, _D), jnp.float32)],
        scratch_types=[
            pltpu.VMEM((_CH,), jnp.int32),
            pltpu.VMEM((_CH,), jnp.int32),
            pltpu.VMEM((_CH, _D), jnp.float32),
            pltpu.VMEM_SHARED((_NP, _D), jnp.float32),
            pltpu.SemaphoreType.DMA,
        ],
    )(functools.partial(_agg_body, False))


_R = 2048


def _tc_layer_body(relu, want_norm, agg_ref, cnt_ref, h_ref, wl_ref, wr_ref,
                   b_ref, z_ref, *nz_ref):
    a = agg_ref[0] + agg_ref[1]
    c = cnt_ref[0] + cnt_ref[1]
    mean = a * (1.0 / jnp.maximum(c, 1.0))
    dn = (((1,), (1,)), ((), ()))
    z = (lax.dot_general(mean, wl_ref[...], dn,
                         precision=lax.Precision.HIGHEST,
                         preferred_element_type=jnp.float32)
         + lax.dot_general(h_ref[...], wr_ref[...], dn,
                           precision=lax.Precision.HIGHEST,
                           preferred_element_type=jnp.float32)
         + b_ref[...])
    if relu:
        z = jnp.maximum(z, 0.0)
    z_ref[...] = z
    if want_norm:
        nz_ref[0][...] = jnp.sqrt(jnp.sum(z * z, axis=1, keepdims=True))


def _tc_layer(aggp, cntp, hin, Wl, Wr, b, relu, want_norm):
    grid = (_NP // _R,)
    out_shape = [jax.ShapeDtypeStruct((_NP, _D), jnp.float32)]
    out_specs = [pl.BlockSpec((_R, _D), lambda i: (i, 0))]
    if want_norm:
        out_shape.append(jax.ShapeDtypeStruct((_NP, 1), jnp.float32))
        out_specs.append(pl.BlockSpec((_R, 1), lambda i: (i, 0)))
    return pl.pallas_call(
        functools.partial(_tc_layer_body, relu, want_norm),
        grid=grid,
        in_specs=[
            pl.BlockSpec((2, _R, _D), lambda i: (0, i, 0)),
            pl.BlockSpec((2, _R, 1), lambda i: (0, i, 0)),
            pl.BlockSpec((_R, _D), lambda i: (i, 0)),
            pl.BlockSpec((_D, _D), lambda i: (0, 0)),
            pl.BlockSpec((_D, _D), lambda i: (0, 0)),
            pl.BlockSpec((1, _D), lambda i: (0, 0)),
        ],
        out_specs=out_specs,
        out_shape=out_shape,
    )(aggp, cntp, hin, Wl, Wr, b)


def _gather_pairs_body(z_hbm, ia_hbm, ib_hbm, za_out, zb_out,
                       aidx, bidx, za, zb, sem):
    cid = lax.axis_index("c")
    sid = lax.axis_index("s")
    wid = sid * 2 + cid
    base0 = wid * (_DC * _CH)

    def chunk(k, carry):
        b0 = base0 + k * _CH
        pltpu.sync_copy(ia_hbm.at[pl.ds(b0, _CH)], aidx)
        pltpu.sync_copy(ib_hbm.at[pl.ds(b0, _CH)], bidx)
        c1 = pltpu.async_copy(z_hbm.at[aidx], za, sem)
        c2 = pltpu.async_copy(z_hbm.at[bidx], zb, sem)
        c1.wait()
        c2.wait()
        pltpu.sync_copy(za, za_out.at[pl.ds(b0, _CH)])
        pltpu.sync_copy(zb, zb_out.at[pl.ds(b0, _CH)])
        return carry

    lax.fori_loop(0, _DC, chunk, 0)


@functools.lru_cache(maxsize=None)
def _gather_pairs_call():
    return functools.partial(
        pl.kernel,
        mesh=_mesh(),
        out_type=[jax.ShapeDtypeStruct((_ELP, _D), jnp.float32),
                  jax.ShapeDtypeStruct((_ELP, _D), jnp.float32)],
        scratch_types=[
            pltpu.VMEM((_CH,), jnp.int32),
            pltpu.VMEM((_CH,), jnp.int32),
            pltpu.VMEM((_CH, _D), jnp.float32),
            pltpu.VMEM((_CH, _D), jnp.float32),
            pltpu.SemaphoreType.DMA,
        ],
    )(_gather_pairs_body)


_RD = 2048


def _cosine_body(za_ref, zb_ref, o_ref):
    za = za_ref[...]
    zb = zb_ref[...]
    num = jnp.sum(za * zb, axis=1, keepdims=True)
    sa = jnp.sum(za * za, axis=1, keepdims=True)
    sb = jnp.sum(zb * zb, axis=1, keepdims=True)
    den = jnp.maximum(jnp.sqrt(sa) * jnp.sqrt(sb), 1e-8)
    o_ref[...] = num / den


def _cosine(za, zb):
    return pl.pallas_call(
        _cosine_body,
        grid=(_ELP // _RD,),
        in_specs=[pl.BlockSpec((_RD, _D), lambda i: (i, 0)),
                  pl.BlockSpec((_RD, _D), lambda i: (i, 0))],
        out_specs=pl.BlockSpec((_RD, 1), lambda i: (i, 0)),
        out_shape=jax.ShapeDtypeStruct((_ELP, 1), jnp.float32),
    )(za, zb)


def kernel(x, edge_index, edge_label_index, W1l, W1r, b1, W2l, W2r, b2):
    src = edge_index[0]
    dst = edge_index[1]
    srcp = jnp.concatenate([src, jnp.zeros((_EP - _E,), jnp.int32)])
    dstp = jnp.concatenate([dst, jnp.full((_EP - _E,), _N, jnp.int32)])
    xp = jnp.concatenate([x, jnp.zeros((_NP - _N, _D), jnp.float32)], axis=0)
    zr = jnp.zeros((_ROWS_PER_TILE, _D), jnp.float32)
    zc = jnp.zeros((_ROWS_PER_TILE,), jnp.float32)

    agg1, cnt = _agg_count_call()(xp, srcp, dstp, zr, zc)
    cnt3 = cnt.reshape(2, _NP, 1)
    h = _tc_layer(agg1, cnt3, xp, W1l, W1r, b1.reshape(1, _D),
                  relu=True, want_norm=False)[0]
    (agg2,) = _agg_call()(h, srcp, dstp, zr)
    (z,) = _tc_layer(agg2, cnt3, h, W2l, W2r, b2.reshape(1, _D),
                     relu=False, want_norm=False)

    ea = jnp.concatenate([edge_label_index[0],
                          jnp.zeros((_ELP - _EL,), jnp.int32)])
    eb = jnp.concatenate([edge_label_index[1],
                          jnp.zeros((_ELP - _EL,), jnp.int32)])
    za, zb = _gather_pairs_call()(z, ea, eb)
    out = _cosine(za, zb)
    return out.reshape(_ELP)[:_EL]
